# baseline (speedup 1.0000x reference)
import functools

import jax
import jax.numpy as jnp
from jax import lax
from jax.experimental import pallas as pl
from jax.experimental.pallas import tpu as pltpu
from jax.experimental.pallas import tpu_sc as plsc

NC = 2
NS = 16
NW = NC * NS
G = 128
NBUF = 2


@functools.partial(jax.jit, static_argnums=(2, 3))
def _gather_sc(g1, tables_flat, rows, d):
    ngroups = rows // G
    npw = ngroups // NW

    mesh = plsc.VectorSubcoreMesh(core_axis_name="c", subcore_axis_name="s")

    @functools.partial(
        pl.kernel,
        out_type=jax.ShapeDtypeStruct((rows, d), jnp.float32),
        mesh=mesh,
        compiler_params=pltpu.CompilerParams(use_tc_tiling_on_sc=False),
        scratch_types=[
            pltpu.VMEM((npw * G,), jnp.int32),
            pltpu.VMEM((NBUF, G, d), jnp.float32),
            pltpu.SemaphoreType.DMA,
            pltpu.SemaphoreType.DMA,
        ],
    )
    def sc_kernel(g_hbm, tab_hbm, out_hbm, idx_v, rows_v, sem0, sem1):
        sems = [sem0, sem1]
        wid = lax.axis_index("s") * NC + lax.axis_index("c")
        g0 = wid * npw

        pltpu.sync_copy(g_hbm.at[pl.ds(g0 * G, npw * G)], idx_v)

        for b in range(NBUF):
            pltpu.async_copy(
                tab_hbm.at[idx_v.at[pl.ds(b * G, G)]], rows_v.at[b], sems[b])

        @pl.loop(0, npw, step=NBUF)
        def _(j0):
            for b in range(NBUF):
                j = j0 + b
                pltpu.make_async_copy(
                    tab_hbm.at[idx_v.at[pl.ds(j * G, G)]],
                    rows_v.at[b], sems[b]).wait()
                pltpu.sync_copy(
                    rows_v.at[b], out_hbm.at[pl.ds((g0 + j) * G, G)])
                @pl.when(j + NBUF < npw)
                def _():
                    pltpu.async_copy(
                        tab_hbm.at[idx_v.at[pl.ds((j + NBUF) * G, G)]],
                        rows_v.at[b], sems[b])

    return sc_kernel(g1, tables_flat)


def kernel(indices, tables):
    f, b = indices.shape
    _, v, d = tables.shape
    rows = f * b
    assert rows % (NW * G) == 0

    tables_flat = tables.reshape(f * v, d)
    offs = (jnp.arange(f, dtype=jnp.int32) * v)[None, :]
    g1 = (indices.T + offs).reshape(rows)

    out = _gather_sc(g1, tables_flat, rows, d)
    return out.reshape(b, f, d)
